# baseline (speedup 1.0000x reference)
import jax
import jax.numpy as jnp
from jax import lax
from jax.experimental import pallas as pl
from jax.experimental.pallas import tpu as pltpu
from jax.experimental.pallas import tpu_sc as plsc

B = 16384
D = 128
L = 16
NC = 2
NS = 16
NW = NC * NS
BPW = B // NW
CHUNK_ROWS = 128
CHUNK_B = CHUNK_ROWS // 2
NCHUNK = BPW // CHUNK_B
GROUPS = CHUNK_B // L


def _body(idx_hbm, table_hbm, w_hbm, b_hbm, out_hbm,
          idx_v, rows_a, rows_b, w_v, b_v, out_v, sem0, sem1):
    wid = lax.axis_index("s") * NC + lax.axis_index("c")
    base = wid * BPW

    pltpu.sync_copy(idx_hbm.at[pl.ds(wid * NCHUNK, NCHUNK)], idx_v)
    pltpu.sync_copy(w_hbm, w_v)
    pltpu.sync_copy(b_hbm, b_v)

    bufs = (rows_a, rows_b)
    sems = (sem0, sem1)

    def start(c):
        return pltpu.async_copy(
            table_hbm.at[idx_v.at[c]], bufs[c % 2], sems[c % 2])

    iota = lax.iota(jnp.int32, L)
    bvec = b_v[...]
    zero = jnp.zeros((L,), jnp.float32)

    grows = [(iota * 2 + g * 2 * L, iota * 2 + g * 2 * L + 1)
             for g in range(GROUPS)]
    dv0_init = jnp.zeros((L,), jnp.int32)

    descs = [None, None]
    descs[0] = start(0)
    for c in range(NCHUNK):
        if c + 1 < NCHUNK:
            descs[(c + 1) % 2] = start(c + 1)
        descs[c % 2].wait()
        buf = bufs[c % 2]

        def dbody(i, carry, buf=buf):
            dv0, accs = carry
            dv1 = dv0 + 1
            w00 = plsc.load_gather(w_v, [dv0])
            w01 = plsc.load_gather(w_v, [dv1])
            w10 = plsc.load_gather(w_v, [dv0 + D])
            w11 = plsc.load_gather(w_v, [dv1 + D])
            new_accs = []
            for g in range(GROUPS):
                a00, a01, a10, a11 = accs[g]
                jrow0, jrow1 = grows[g]
                g00 = plsc.load_gather(buf, [jrow0, dv0])
                g01 = plsc.load_gather(buf, [jrow0, dv1])
                g10 = plsc.load_gather(buf, [jrow1, dv0])
                g11 = plsc.load_gather(buf, [jrow1, dv1])
                new_accs.append((a00 + g00 * w00, a01 + g01 * w01,
                                 a10 + g10 * w10, a11 + g11 * w11))
            return (dv0 + 2, tuple(new_accs))

        init = (dv0_init, tuple((zero, zero, zero, zero)
                                for _ in range(GROUPS)))
        _, accs = plsc.parallel_loop(
            0, D // 2, 1, unroll=4, carry=init)(dbody)
        for g in range(GROUPS):
            a00, a01, a10, a11 = accs[g]
            z = (a00 + a01) + (a10 + a11) + bvec
            out_v[pl.ds(c * CHUNK_B + g * L, L)] = 1.0 / (1.0 + jnp.exp(-z))

    pltpu.sync_copy(out_v, out_hbm.at[pl.ds(base, BPW)])


_sc_call = pl.kernel(
    _body,
    out_type=jax.ShapeDtypeStruct((B,), jnp.float32),
    mesh=plsc.VectorSubcoreMesh(core_axis_name="c", subcore_axis_name="s"),
    scratch_types=[
        pltpu.VMEM((NCHUNK, CHUNK_ROWS), jnp.int32),
        pltpu.VMEM((CHUNK_ROWS, D), jnp.float32),
        pltpu.VMEM((CHUNK_ROWS, D), jnp.float32),
        pltpu.VMEM((2 * D,), jnp.float32),
        pltpu.VMEM((L,), jnp.float32),
        pltpu.VMEM((BPW,), jnp.float32),
        pltpu.SemaphoreType.DMA,
        pltpu.SemaphoreType.DMA,
    ],
    compiler_params=pltpu.CompilerParams(needs_layout_passes=False),
)


@jax.jit
def kernel(x, emb_table, W, b):
    idx = x.astype(jnp.int32).reshape(NW * NCHUNK, CHUNK_ROWS)
    w = W.reshape(2 * D).astype(jnp.float32)
    b16 = jnp.broadcast_to(b.reshape(()), (L,)).astype(jnp.float32)
    out = _sc_call(idx, emb_table, w, b16)
    return out.reshape(B, 1)


# speedup vs baseline: 2.2546x; 2.2546x over previous
import jax
import jax.numpy as jnp
from jax import lax
from jax.experimental import pallas as pl
from jax.experimental.pallas import tpu as pltpu
from jax.experimental.pallas import tpu_sc as plsc

B = 16384
D = 128
L = 16
NC = 2
NS = 16
NW = NC * NS
BPW = B // NW
CHUNK_ROWS = 128
CHUNK_B = CHUNK_ROWS // 2
NCHUNK = BPW // CHUNK_B
GROUPS = CHUNK_B // L


def _body(idx_hbm, table_hbm, w_hbm, b_hbm, out_hbm,
          idx_v, rows_a, rows_b, w_v, b_v, out_v, sem0, sem1):
    wid = lax.axis_index("s") * NC + lax.axis_index("c")
    base = wid * BPW

    pltpu.sync_copy(idx_hbm.at[pl.ds(wid * NCHUNK, NCHUNK)], idx_v)
    pltpu.sync_copy(w_hbm, w_v)
    pltpu.sync_copy(b_hbm, b_v)

    bufs = (rows_a, rows_b)
    sems = (sem0, sem1)

    def start(c):
        return pltpu.async_copy(
            table_hbm.at[idx_v.at[c]], bufs[c % 2], sems[c % 2])

    iota = lax.iota(jnp.int32, L)
    lane0 = iota == 0
    bvec = b_v[...]
    zero = jnp.zeros((L,), jnp.float32)

    w_regs = [w_v[pl.ds(k * L, L)] for k in range(2 * D // L)]
    KPR = D // L

    descs = [None, None]
    descs[0] = start(0)
    for c in range(NCHUNK):
        if c + 1 < NCHUNK:
            descs[(c + 1) % 2] = start(c + 1)
        descs[c % 2].wait()
        buf = bufs[c % 2]

        def jbody(j, buf=buf, c=c):
            r0 = j * 2
            r1 = r0 + 1
            accs = [bvec, zero, zero, zero]
            for k in range(KPR):
                accs[k % 4] = accs[k % 4] + buf[r0, pl.ds(k * L, L)] * w_regs[k]
            for k in range(KPR):
                accs[k % 4] = (accs[k % 4]
                               + buf[r1, pl.ds(k * L, L)] * w_regs[KPR + k])
            s = jnp.sum((accs[0] + accs[1]) + (accs[2] + accs[3]))
            sv = jnp.full((L,), s, jnp.float32)
            res = 1.0 / (1.0 + jnp.exp(-sv))
            pos = jnp.full((L,), j + c * CHUNK_B, jnp.int32)
            plsc.store_scatter(out_v, [pos], res, mask=lane0)

        plsc.parallel_loop(0, CHUNK_B, 1, unroll=4)(jbody)

    pltpu.sync_copy(out_v, out_hbm.at[pl.ds(base, BPW)])


_sc_call = pl.kernel(
    _body,
    out_type=jax.ShapeDtypeStruct((B,), jnp.float32),
    mesh=plsc.VectorSubcoreMesh(core_axis_name="c", subcore_axis_name="s"),
    scratch_types=[
        pltpu.VMEM((NCHUNK, CHUNK_ROWS), jnp.int32),
        pltpu.VMEM((CHUNK_ROWS, D), jnp.float32),
        pltpu.VMEM((CHUNK_ROWS, D), jnp.float32),
        pltpu.VMEM((2 * D,), jnp.float32),
        pltpu.VMEM((L,), jnp.float32),
        pltpu.VMEM((BPW,), jnp.float32),
        pltpu.SemaphoreType.DMA,
        pltpu.SemaphoreType.DMA,
    ],
    compiler_params=pltpu.CompilerParams(needs_layout_passes=False),
)


@jax.jit
def kernel(x, emb_table, W, b):
    idx = x.astype(jnp.int32).reshape(NW * NCHUNK, CHUNK_ROWS)
    w = W.reshape(2 * D).astype(jnp.float32)
    b16 = jnp.zeros((L,), jnp.float32).at[0].set(b.reshape(())[()])
    out = _sc_call(idx, emb_table, w, b16)
    return out.reshape(B, 1)


# speedup vs baseline: 2.2819x; 1.0121x over previous
import jax
import jax.numpy as jnp
from jax import lax
from jax.experimental import pallas as pl
from jax.experimental.pallas import tpu as pltpu
from jax.experimental.pallas import tpu_sc as plsc

B = 16384
D = 128
L = 16
NC = 2
NS = 16
NW = NC * NS
BPW = B // NW
CHUNK_ROWS = 128
CHUNK_B = CHUNK_ROWS // 2
NCHUNK = BPW // CHUNK_B
GROUPS = CHUNK_B // L


NBUF = 4


def _body(idx_hbm, table_hbm, w_hbm, b_hbm, out_hbm,
          idx_v, rows_a, rows_b, rows_c, rows_d, w_v, b_v, out_v,
          sem0, sem1, sem2, sem3):
    wid = lax.axis_index("s") * NC + lax.axis_index("c")
    base = wid * BPW

    pltpu.sync_copy(idx_hbm.at[pl.ds(wid * NCHUNK, NCHUNK)], idx_v)
    pltpu.sync_copy(w_hbm, w_v)
    pltpu.sync_copy(b_hbm, b_v)

    bufs = (rows_a, rows_b, rows_c, rows_d)
    sems = (sem0, sem1, sem2, sem3)

    def start(c):
        return pltpu.async_copy(
            table_hbm.at[idx_v.at[c]], bufs[c % NBUF], sems[c % NBUF])

    iota = lax.iota(jnp.int32, L)
    lane0 = iota == 0
    bvec = b_v[...]
    zero = jnp.zeros((L,), jnp.float32)

    w_regs = [w_v[pl.ds(k * L, L)] for k in range(2 * D // L)]
    KPR = D // L

    descs = [None] * NBUF
    for c in range(NBUF - 1):
        descs[c] = start(c)
    for c in range(NCHUNK):
        nxt = c + NBUF - 1
        if nxt < NCHUNK:
            descs[nxt % NBUF] = start(nxt)
        descs[c % NBUF].wait()
        buf = bufs[c % NBUF]

        def jbody(j, buf=buf, c=c):
            r0 = j * 2
            r1 = r0 + 1
            accs = [bvec, zero, zero, zero]
            for k in range(KPR):
                accs[k % 4] = accs[k % 4] + buf[r0, pl.ds(k * L, L)] * w_regs[k]
            for k in range(KPR):
                accs[k % 4] = (accs[k % 4]
                               + buf[r1, pl.ds(k * L, L)] * w_regs[KPR + k])
            s = jnp.sum((accs[0] + accs[1]) + (accs[2] + accs[3]))
            sv = jnp.full((L,), s, jnp.float32)
            res = 1.0 / (1.0 + jnp.exp(-sv))
            pos = jnp.full((L,), j + c * CHUNK_B, jnp.int32)
            plsc.store_scatter(out_v, [pos], res, mask=lane0)

        plsc.parallel_loop(0, CHUNK_B, 1, unroll=4)(jbody)

    pltpu.sync_copy(out_v, out_hbm.at[pl.ds(base, BPW)])


_sc_call = pl.kernel(
    _body,
    out_type=jax.ShapeDtypeStruct((B,), jnp.float32),
    mesh=plsc.VectorSubcoreMesh(core_axis_name="c", subcore_axis_name="s"),
    scratch_types=[
        pltpu.VMEM((NCHUNK, CHUNK_ROWS), jnp.int32),
        pltpu.VMEM((CHUNK_ROWS, D), jnp.float32),
        pltpu.VMEM((CHUNK_ROWS, D), jnp.float32),
        pltpu.VMEM((CHUNK_ROWS, D), jnp.float32),
        pltpu.VMEM((CHUNK_ROWS, D), jnp.float32),
        pltpu.VMEM((2 * D,), jnp.float32),
        pltpu.VMEM((L,), jnp.float32),
        pltpu.VMEM((BPW,), jnp.float32),
        pltpu.SemaphoreType.DMA,
        pltpu.SemaphoreType.DMA,
        pltpu.SemaphoreType.DMA,
        pltpu.SemaphoreType.DMA,
    ],
    compiler_params=pltpu.CompilerParams(needs_layout_passes=False),
)


@jax.jit
def kernel(x, emb_table, W, b):
    idx = x.astype(jnp.int32).reshape(NW * NCHUNK, CHUNK_ROWS)
    w = W.reshape(2 * D).astype(jnp.float32)
    b16 = jnp.zeros((L,), jnp.float32).at[0].set(b.reshape(())[()])
    out = _sc_call(idx, emb_table, w, b16)
    return out.reshape(B, 1)


# speedup vs baseline: 2.3574x; 1.0331x over previous
import jax
import jax.numpy as jnp
from jax import lax
from jax.experimental import pallas as pl
from jax.experimental.pallas import tpu as pltpu
from jax.experimental.pallas import tpu_sc as plsc

B = 16384
D = 128
L = 16
NC = 2
NS = 16
NW = NC * NS
BPW = B // NW
CHUNK_ROWS = 128
CHUNK_B = CHUNK_ROWS // 2
NCHUNK = BPW // CHUNK_B
GROUPS = CHUNK_B // L


NBUF = 4


def _body(idx_hbm, table_hbm, w_hbm, b_hbm, out_hbm,
          idx_v, rows_a, rows_b, rows_c, rows_d, w_v, b_v, out_v,
          sem0, sem1, sem2, sem3, sem_out):
    wid = lax.axis_index("s") * NC + lax.axis_index("c")
    base = wid * BPW

    d_idx = pltpu.async_copy(
        idx_hbm.at[pl.ds(wid * NCHUNK, NCHUNK)], idx_v, sem0)
    d_w = pltpu.async_copy(w_hbm, w_v, sem1)
    d_b = pltpu.async_copy(b_hbm, b_v, sem2)
    d_idx.wait()
    d_w.wait()
    d_b.wait()

    bufs = (rows_a, rows_b, rows_c, rows_d)
    sems = (sem0, sem1, sem2, sem3)

    def start(c):
        return pltpu.async_copy(
            table_hbm.at[idx_v.at[c]], bufs[c % NBUF], sems[c % NBUF])

    iota = lax.iota(jnp.int32, L)
    lane0 = iota == 0
    bvec = b_v[...]
    zero = jnp.zeros((L,), jnp.float32)

    w_regs = [w_v[pl.ds(k * L, L)] for k in range(2 * D // L)]
    KPR = D // L

    descs = [None] * NBUF
    out_descs = []
    for c in range(NBUF - 1):
        descs[c] = start(c)
    for c in range(NCHUNK):
        nxt = c + NBUF - 1
        if nxt < NCHUNK:
            descs[nxt % NBUF] = start(nxt)
        descs[c % NBUF].wait()
        buf = bufs[c % NBUF]

        def jbody(j, buf=buf, c=c):
            r0 = j * 2
            r1 = r0 + 1
            accs = [bvec, zero, zero, zero]
            for k in range(KPR):
                accs[k % 4] = accs[k % 4] + buf[r0, pl.ds(k * L, L)] * w_regs[k]
            for k in range(KPR):
                accs[k % 4] = (accs[k % 4]
                               + buf[r1, pl.ds(k * L, L)] * w_regs[KPR + k])
            s = jnp.sum((accs[0] + accs[1]) + (accs[2] + accs[3]))
            sv = jnp.full((L,), s, jnp.float32)
            res = 1.0 / (1.0 + jnp.exp(-sv))
            pos = jnp.full((L,), j + c * CHUNK_B, jnp.int32)
            plsc.store_scatter(out_v, [pos], res, mask=lane0)

        plsc.parallel_loop(0, CHUNK_B, 1, unroll=4)(jbody)

        out_descs.append(pltpu.async_copy(
            out_v.at[pl.ds(c * CHUNK_B, CHUNK_B)],
            out_hbm.at[pl.ds(base + c * CHUNK_B, CHUNK_B)], sem_out))

    for d in out_descs:
        d.wait()


_sc_call = pl.kernel(
    _body,
    out_type=jax.ShapeDtypeStruct((B,), jnp.float32),
    mesh=plsc.VectorSubcoreMesh(core_axis_name="c", subcore_axis_name="s"),
    scratch_types=[
        pltpu.VMEM((NCHUNK, CHUNK_ROWS), jnp.int32),
        pltpu.VMEM((CHUNK_ROWS, D), jnp.float32),
        pltpu.VMEM((CHUNK_ROWS, D), jnp.float32),
        pltpu.VMEM((CHUNK_ROWS, D), jnp.float32),
        pltpu.VMEM((CHUNK_ROWS, D), jnp.float32),
        pltpu.VMEM((2 * D,), jnp.float32),
        pltpu.VMEM((L,), jnp.float32),
        pltpu.VMEM((BPW,), jnp.float32),
        pltpu.SemaphoreType.DMA,
        pltpu.SemaphoreType.DMA,
        pltpu.SemaphoreType.DMA,
        pltpu.SemaphoreType.DMA,
        pltpu.SemaphoreType.DMA,
    ],
    compiler_params=pltpu.CompilerParams(needs_layout_passes=False),
)


@jax.jit
def kernel(x, emb_table, W, b):
    idx = x.astype(jnp.int32).reshape(NW * NCHUNK, CHUNK_ROWS)
    w = W.reshape(2 * D).astype(jnp.float32)
    b16 = jnp.zeros((L,), jnp.float32).at[0].set(b.reshape(())[()])
    out = _sc_call(idx, emb_table, w, b16)
    return out.reshape(B, 1)
